# baseline (speedup 1.0000x reference)
import functools

import jax
import jax.numpy as jnp
from jax import lax
from jax.experimental import pallas as pl
from jax.experimental.pallas import tpu as pltpu
from jax.experimental.pallas import tpu_sc as plsc

N = 10000
E = 160000
D = 256
DH = D // 2
NS = 16
NC = 2
CH = 64
EPW = E // NS
NFULL = EPW // CH
TAIL = EPW - NFULL * CH
NCHUNK = NFULL + 1
EPW_PAD = NCHUNK * CH
SRCROWS = NCHUNK // 2 + 1
ACC_ROWS = 10112
ZPW = ACC_ROWS // NS


def _sc_edge_aggregate(nfeat, efeat, srcs, dsts, zrows):
    mesh = plsc.VectorSubcoreMesh(core_axis_name="c", subcore_axis_name="s")

    @functools.partial(
        pl.kernel,
        out_type=jax.ShapeDtypeStruct((ACC_ROWS, D), jnp.float32),
        mesh=mesh,
        scratch_types=[
            pltpu.VMEM((SRCROWS, 2 * CH), jnp.int32),
            pltpu.VMEM((1, CH), jnp.int32),
            pltpu.VMEM((1, CH), jnp.int32),
            pltpu.VMEM((CH, DH), jnp.float32),
            pltpu.VMEM((CH, DH), jnp.float32),
            pltpu.VMEM((CH, DH), jnp.float32),
            pltpu.VMEM((CH, DH), jnp.float32),
            pltpu.SemaphoreType.DMA,
            pltpu.SemaphoreType.DMA,
            pltpu.VMEM_SHARED((ACC_ROWS, DH), jnp.float32),
        ],
    )
    def k(nfeat_hbm, efeat_hbm, src_hbm, dst_hbm, zero_hbm, out_hbm,
          src_v, d0, d1, g0, g1, e0, e1, sem0, sem1, acc):
        c = lax.axis_index("c")
        s = lax.axis_index("s")
        col = c * DH
        dd = (d0, d1)
        gg = (g0, g1)
        ee = (e0, e1)
        ss = (sem0, sem1)

        pltpu.sync_copy(src_hbm.at[s], src_v)

        def issue(b, j):
            pltpu.async_copy(
                nfeat_hbm.at[src_v.at[(j - b) // 2, pl.ds(b * CH, CH)],
                             pl.ds(col, DH)],
                gg[b], ss[b])
            pltpu.async_copy(
                efeat_hbm.at[pl.ds(s * EPW + j * CH, CH), pl.ds(col, DH)],
                ee[b], ss[b])
            pltpu.async_copy(dst_hbm.at[s, pl.ds(j, 1)], dd[b], ss[b])

        def drain(b):
            pltpu.make_async_copy(zero_hbm.at[pl.ds(0, CH)], gg[b],
                                  ss[b]).wait()
            pltpu.make_async_copy(zero_hbm.at[pl.ds(0, CH)], ee[b],
                                  ss[b]).wait()
            pltpu.make_async_copy(dst_hbm.at[s, pl.ds(0, 1)], dd[b],
                                  ss[b]).wait()

        def consume(b):
            gbuf, ebuf = gg[b], ee[b]

            def rows(r, _):
                for kk in range(DH // 16):
                    sl = (r, pl.ds(kk * 16, 16))
                    ebuf[sl] = jnp.maximum(ebuf[sl] + gbuf[sl], 0.0)
                return 0
            lax.fori_loop(0, CH, rows, 0)
            pltpu.sync_copy(ebuf, acc.at[dd[b].at[0]], add=True)

        issue(0, 0)
        issue(1, 1)
        pltpu.sync_copy(zero_hbm, acc.at[pl.ds(s * ZPW, ZPW)])
        plsc.subcore_barrier()

        def pair_body(o, _):
            for b in range(2):
                drain(b)
                consume(b)
                issue(b, 2 * o + b + 2)
            return 0
        lax.fori_loop(0, NFULL // 2 - 1, pair_body, 0)
        for b in range(2):
            drain(b)
            consume(b)

        pltpu.sync_copy(
            nfeat_hbm.at[src_v.at[NFULL // 2, pl.ds(0, CH)], pl.ds(col, DH)],
            g0)
        pltpu.sync_copy(dst_hbm.at[s, pl.ds(NFULL, 1)], d0)
        pltpu.sync_copy(
            efeat_hbm.at[pl.ds(s * EPW + NFULL * CH, TAIL), pl.ds(col, DH)],
            e0.at[pl.ds(0, TAIL)])
        consume(0)

        plsc.subcore_barrier()
        pltpu.sync_copy(acc.at[pl.ds(s * ZPW, ZPW)],
                        out_hbm.at[pl.ds(s * ZPW, ZPW), pl.ds(col, DH)])

    return k(nfeat, efeat, srcs, dsts, zrows)


def _tc_pre(nfeat, W1, b1):
    def body(nf_r, w1_r, b1_r, out_r):
        out_r[...] = lax.dot_general(nf_r[...], w1_r[...],
                                     (((1,), (1,)), ((), ())),
                                     preferred_element_type=jnp.float32) + b1_r[...]

    return pl.pallas_call(
        body,
        out_shape=jax.ShapeDtypeStruct((N, 2 * D), jnp.float32),
    )(nfeat, W1, b1.reshape(1, 2 * D))


def _tc_mlp(agg, P, W1, gamma, beta, W2, b2):
    def body(agg_r, w1_r, p_r, g_r, bb_r, w2_r, b2_r, out_r):
        h1 = lax.dot_general(agg_r[0:N, :], w1_r[...],
                             (((1,), (1,)), ((), ())),
                             preferred_element_type=jnp.float32) + p_r[...]
        mean = jnp.mean(h1, axis=0, keepdims=True)
        var = jnp.mean(jnp.square(h1 - mean), axis=0, keepdims=True)
        hn = (h1 - mean) * lax.rsqrt(var + 1e-5) * g_r[...] + bb_r[...]
        hn = jnp.maximum(hn, 0.0)
        out_r[...] = lax.dot_general(hn, w2_r[...], (((1,), (1,)), ((), ())),
                                     preferred_element_type=jnp.float32) + b2_r[...]

    return pl.pallas_call(
        body,
        out_shape=jax.ShapeDtypeStruct((N, D), jnp.float32),
        compiler_params=pltpu.CompilerParams(
            vmem_limit_bytes=100 * 1024 * 1024),
    )(agg, W1, P, gamma.reshape(1, 2 * D),
      beta.reshape(1, 2 * D), W2, b2.reshape(1, D))


def kernel(nfeat, edge_index, efeat, W1, b1, gamma, beta, W2, b2):
    src = edge_index[0].reshape(NS, EPW)
    dst = edge_index[1].reshape(NS, EPW)
    srcs = jnp.pad(src, ((0, 0), (0, SRCROWS * 2 * CH - EPW)))
    srcs = srcs.reshape(NS, SRCROWS, 2 * CH)
    dsts = jnp.pad(dst, ((0, 0), (0, EPW_PAD - EPW)),
                   constant_values=N).reshape(NS, NCHUNK, CH)

    zrows = jnp.zeros((ZPW, DH), jnp.float32)
    P = _tc_pre(nfeat, W1, b1)
    agg = _sc_edge_aggregate(nfeat, efeat, srcs, dsts, zrows)
    return _tc_mlp(agg, P, W1, gamma, beta, W2, b2)


# speedup vs baseline: 1.0376x; 1.0376x over previous
import functools

import jax
import jax.numpy as jnp
from jax import lax
from jax.experimental import pallas as pl
from jax.experimental.pallas import tpu as pltpu
from jax.experimental.pallas import tpu_sc as plsc

N = 10000
E = 160000
D = 256
DH = D // 2
NS = 16
NC = 2
CH = 64
EPW = E // NS
NFULL = EPW // CH
TAIL = EPW - NFULL * CH
NCHUNK = NFULL + 1
EPW_PAD = NCHUNK * CH
SRCROWS = NCHUNK // 2 + 1
ACC_ROWS = 10112
ZPW = ACC_ROWS // NS


def _sc_edge_aggregate(nfeat, efeat, srcs, dsts, zrows):
    mesh = plsc.VectorSubcoreMesh(core_axis_name="c", subcore_axis_name="s")

    @functools.partial(
        pl.kernel,
        out_type=jax.ShapeDtypeStruct((ACC_ROWS, D), jnp.float32),
        mesh=mesh,
        scratch_types=[
            pltpu.VMEM((SRCROWS, 2 * CH), jnp.int32),
            pltpu.VMEM((1, CH), jnp.int32),
            pltpu.VMEM((1, CH), jnp.int32),
            pltpu.VMEM((CH, DH), jnp.float32),
            pltpu.VMEM((CH, DH), jnp.float32),
            pltpu.VMEM((CH, DH), jnp.float32),
            pltpu.VMEM((CH, DH), jnp.float32),
            pltpu.SemaphoreType.DMA,
            pltpu.SemaphoreType.DMA,
            pltpu.VMEM_SHARED((ACC_ROWS, DH), jnp.float32),
        ],
    )
    def k(nfeat_hbm, efeat_hbm, src_hbm, dst_hbm, zero_hbm, out_hbm,
          src_v, d0, d1, g0, g1, e0, e1, sem0, sem1, acc):
        c = lax.axis_index("c")
        s = lax.axis_index("s")
        col = c * DH
        dd = (d0, d1)
        gg = (g0, g1)
        ee = (e0, e1)
        ss = (sem0, sem1)

        pltpu.sync_copy(src_hbm.at[s], src_v)

        def issue(b, j):
            pltpu.async_copy(
                nfeat_hbm.at[src_v.at[(j - b) // 2, pl.ds(b * CH, CH)],
                             pl.ds(col, DH)],
                gg[b], ss[b])
            pltpu.async_copy(
                efeat_hbm.at[pl.ds(s * EPW + j * CH, CH), pl.ds(col, DH)],
                ee[b], ss[b])
            pltpu.async_copy(dst_hbm.at[s, pl.ds(j, 1)], dd[b], ss[b])

        def drain(b):
            pltpu.make_async_copy(zero_hbm.at[pl.ds(0, CH)], gg[b],
                                  ss[b]).wait()
            pltpu.make_async_copy(zero_hbm.at[pl.ds(0, CH)], ee[b],
                                  ss[b]).wait()
            pltpu.make_async_copy(dst_hbm.at[s, pl.ds(0, 1)], dd[b],
                                  ss[b]).wait()

        def consume(b):
            gbuf, ebuf = gg[b], ee[b]

            def rows(r, _):
                for kk in range(DH // 16):
                    sl = (r, pl.ds(kk * 16, 16))
                    ebuf[sl] = jnp.maximum(ebuf[sl] + gbuf[sl], 0.0)
                return 0
            lax.fori_loop(0, CH, rows, 0)
            pltpu.sync_copy(ebuf, acc.at[dd[b].at[0]], add=True)

        issue(0, 0)
        issue(1, 1)
        pltpu.sync_copy(zero_hbm, acc.at[pl.ds(s * ZPW, ZPW)])
        plsc.subcore_barrier()

        def pair_body(o, _):
            for b in range(2):
                drain(b)
                consume(b)
                issue(b, 2 * o + b + 2)
            return 0
        lax.fori_loop(0, NFULL // 2 - 1, pair_body, 0)
        for b in range(2):
            drain(b)
            consume(b)

        pltpu.sync_copy(
            nfeat_hbm.at[src_v.at[NFULL // 2, pl.ds(0, CH)], pl.ds(col, DH)],
            g0)
        pltpu.sync_copy(dst_hbm.at[s, pl.ds(NFULL, 1)], d0)
        pltpu.sync_copy(
            efeat_hbm.at[pl.ds(s * EPW + NFULL * CH, TAIL), pl.ds(col, DH)],
            e0.at[pl.ds(0, TAIL)])
        consume(0)

        plsc.subcore_barrier()
        pltpu.sync_copy(acc.at[pl.ds(s * ZPW, ZPW)],
                        out_hbm.at[pl.ds(s * ZPW, ZPW), pl.ds(col, DH)])

    return k(nfeat, efeat, srcs, dsts, zrows)


def _tc_mlp(agg, nfeat, W1, b1, gamma, beta, W2, b2):
    def body(agg_r, nf_r, w1_r, b1_r, g_r, bb_r, w2_r, b2_r, out_r):
        h = agg_r[0:N, :] + nf_r[...]
        h1 = lax.dot_general(h, w1_r[...], (((1,), (1,)), ((), ())),
                             preferred_element_type=jnp.float32) + b1_r[...]
        mean = jnp.mean(h1, axis=0, keepdims=True)
        var = jnp.mean(jnp.square(h1 - mean), axis=0, keepdims=True)
        hn = (h1 - mean) * lax.rsqrt(var + 1e-5) * g_r[...] + bb_r[...]
        hn = jnp.maximum(hn, 0.0)
        out_r[...] = lax.dot_general(hn, w2_r[...], (((1,), (1,)), ((), ())),
                                     preferred_element_type=jnp.float32) + b2_r[...]

    return pl.pallas_call(
        body,
        out_shape=jax.ShapeDtypeStruct((N, D), jnp.float32),
    )(agg, nfeat, W1, b1.reshape(1, 2 * D), gamma.reshape(1, 2 * D),
      beta.reshape(1, 2 * D), W2, b2.reshape(1, D))


def kernel(nfeat, edge_index, efeat, W1, b1, gamma, beta, W2, b2):
    src = edge_index[0].reshape(NS, EPW)
    dst = edge_index[1].reshape(NS, EPW)
    srcs = jnp.pad(src, ((0, 0), (0, SRCROWS * 2 * CH - EPW)))
    srcs = srcs.reshape(NS, SRCROWS, 2 * CH)
    dsts = jnp.pad(dst, ((0, 0), (0, EPW_PAD - EPW)),
                   constant_values=N).reshape(NS, NCHUNK, CH)

    zrows = jnp.zeros((ZPW, DH), jnp.float32)
    agg = _sc_edge_aggregate(nfeat, efeat, srcs, dsts, zrows)
    return _tc_mlp(agg, nfeat, W1, b1, gamma, beta, W2, b2)
